# baseline (speedup 1.0000x reference)
import functools

import jax
import jax.numpy as jnp
from jax import lax
from jax.experimental import pallas as pl
from jax.experimental.pallas import tpu as pltpu
from jax.experimental.pallas import tpu_sc as plsc

N_NODES = 10000
D = 128
BN_EPS = 1e-5

NC = 2
NS = 16
NW = NC * NS
CHUNK = 120
ROWS_MAIN = 632
ROWS_LAST = N_NODES - 15 * ROWS_MAIN


def _segment_sum_sc(src_flat, dst_flat, zpad, x, n0, n1):
    e = src_flat.shape[0]
    full = e // CHUNK
    tail = e - full * CHUNK
    mesh = plsc.VectorSubcoreMesh(core_axis_name="c", subcore_axis_name="s")

    @functools.partial(
        pl.kernel,
        mesh=mesh,
        out_type=jax.ShapeDtypeStruct((NC, N_NODES, D), jnp.float32),
        scratch_types=[
            pltpu.VMEM((6, CHUNK), jnp.int32),
            pltpu.VMEM((6, CHUNK), jnp.int32),
            pltpu.VMEM((CHUNK, D), jnp.float32),
            pltpu.VMEM((CHUNK, D), jnp.float32),
            pltpu.VMEM((CHUNK, D), jnp.float32),
            pltpu.VMEM_SHARED((N_NODES, D), jnp.float32),
            pltpu.SemaphoreType.DMA,
            pltpu.SemaphoreType.DMA,
            pltpu.SemaphoreType.DMA,
            pltpu.SemaphoreType.DMA,
            pltpu.SemaphoreType.DMA,
            pltpu.SemaphoreType.DMA,
            pltpu.SemaphoreType.DMA,
            pltpu.SemaphoreType.DMA,
        ],
    )
    def seg_sum(src_hbm, dst_hbm, zpad_hbm, x_hbm, out_hbm, sidx, didx,
                rows0, rows1, rows2, acc,
                semg0, semg1, semg2, sems0, sems1, sems2, semia, semib):
        c = lax.axis_index("c")
        s = lax.axis_index("s")
        n_chunks = jnp.where(c == 0, n0, n1)
        n_super = n_chunks // 6

        bufs = (rows0, rows1, rows2)
        semgs = (semg0, semg1, semg2)
        semss = (sems0, sems1, sems2)

        zero = jnp.zeros((16,), jnp.float32)

        def zrow(i, _):
            def zcol(j, _):
                rows0[i, pl.ds(j * 16, 16)] = zero
                return 0
            return lax.fori_loop(0, D // 16, zcol, 0)

        lax.fori_loop(0, CHUNK, zrow, 0)

        abase = s * ROWS_MAIN

        def zero_span(nrows):
            for i in range(nrows // CHUNK):
                pltpu.sync_copy(rows0, acc.at[pl.ds(abase + i * CHUNK, CHUNK)])
            rem = nrows % CHUNK
            if rem:
                pltpu.sync_copy(
                    rows0.at[pl.ds(0, rem)],
                    acc.at[pl.ds(abase + (nrows // CHUNK) * CHUNK, rem)])

        @pl.when(s < NS - 1)
        def _():
            zero_span(ROWS_MAIN)

        @pl.when(s == NS - 1)
        def _():
            zero_span(ROWS_LAST)

        cbase = jnp.where(c == 0, s * n0, NS * n0 + s * n1)

        def load_idx(arr, row_dst, ch, sem):
            @pl.when(ch < full)
            def _():
                pltpu.async_copy(arr.at[pl.ds(ch * CHUNK, CHUNK)], row_dst, sem)

            if tail:
                @pl.when(ch == full)
                def _():
                    pltpu.async_copy(
                        arr.at[pl.ds(full * CHUNK, tail)],
                        row_dst.at[pl.ds(0, tail)], sem)
                    pltpu.async_copy(
                        zpad_hbm.at[pl.ds(0, CHUNK - tail)],
                        row_dst.at[pl.ds(tail, CHUNK - tail)], sem)

            @pl.when(ch > full if tail else ch >= full)
            def _():
                pltpu.async_copy(zpad_hbm, row_dst, sem)

        def drain_idx(arr, row_dst, ch, sem):
            @pl.when(ch < full)
            def _():
                pltpu.make_async_copy(
                    arr.at[pl.ds(ch * CHUNK, CHUNK)], row_dst, sem).wait()

            if tail:
                @pl.when(ch == full)
                def _():
                    pltpu.make_async_copy(
                        arr.at[pl.ds(full * CHUNK, tail)],
                        row_dst.at[pl.ds(0, tail)], sem).wait()
                    pltpu.make_async_copy(
                        zpad_hbm.at[pl.ds(0, CHUNK - tail)],
                        row_dst.at[pl.ds(tail, CHUNK - tail)], sem).wait()

            @pl.when(ch > full if tail else ch >= full)
            def _():
                pltpu.make_async_copy(zpad_hbm, row_dst, sem).wait()

        for u in range(6):
            load_idx(src_hbm, sidx.at[u], cbase + u, semia)
            load_idx(dst_hbm, didx.at[u], cbase + u, semia)
        for u in range(6):
            drain_idx(src_hbm, sidx.at[u], cbase + u, semia)
            drain_idx(dst_hbm, didx.at[u], cbase + u, semia)
        plsc.subcore_barrier()

        def wait_g(u, b):
            pltpu.make_async_copy(x_hbm.at[sidx.at[u]], bufs[b], semgs[b]).wait()

        def issue_s(u, b):
            pltpu.async_copy(bufs[b], acc.at[didx.at[u]], semss[b], add=True)

        def wait_s(u, b):
            pltpu.make_async_copy(bufs[b], acc.at[didx.at[u]], semss[b]).wait()

        def issue_g(u, b):
            pltpu.async_copy(x_hbm.at[sidx.at[u]], bufs[b], semgs[b])

        def refill(slot0, c0, sem):
            for u in range(3):
                load_idx(src_hbm, sidx.at[slot0 + u], c0 + u, sem)
                load_idx(dst_hbm, didx.at[slot0 + u], c0 + u, sem)

        def wait_refill(slot0, c0, sem):
            for u in range(3):
                drain_idx(src_hbm, sidx.at[slot0 + u], c0 + u, sem)
                drain_idx(dst_hbm, didx.at[slot0 + u], c0 + u, sem)

        def first_half(j0, do_refill_a, wait_ib, issue_gathers):
            for u in range(3):
                wait_g(u, u)
                issue_s(u, u)
            if wait_ib:
                wait_refill(3, cbase + j0 + 3, semib)
            for u in range(3):
                wait_s(u, u)
                if issue_gathers:
                    issue_g(3 + u, u)
            if do_refill_a:
                refill(0, cbase + j0 + 6, semia)

        def second_half(j0, do_refill_b, wait_ia, issue_gathers):
            for u in range(3):
                wait_g(3 + u, u)
                issue_s(3 + u, u)
            if wait_ia:
                wait_refill(0, cbase + j0 + 6, semia)
            for u in range(3):
                wait_s(3 + u, u)
                if issue_gathers:
                    issue_g(u, u)
            if do_refill_b:
                refill(3, cbase + j0 + 9, semib)

        for u in range(3):
            issue_g(u, u)

        first_half(0, do_refill_a=True, wait_ib=False, issue_gathers=True)
        second_half(0, do_refill_b=True, wait_ia=True, issue_gathers=True)

        def body(jj, _):
            j0 = jj * 6
            first_half(j0, True, True, True)
            second_half(j0, True, True, True)
            return 0

        lax.fori_loop(1, n_super - 1, body, 0)

        j0e = (n_super - 1) * 6
        first_half(j0e, do_refill_a=False, wait_ib=True, issue_gathers=True)
        second_half(j0e, do_refill_b=False, wait_ia=False, issue_gathers=False)

        plsc.subcore_barrier()

        @pl.when(s < NS - 1)
        def _():
            pltpu.sync_copy(
                acc.at[pl.ds(abase, ROWS_MAIN)],
                out_hbm.at[c, pl.ds(abase, ROWS_MAIN)],
            )

        @pl.when(s == NS - 1)
        def _():
            pltpu.sync_copy(
                acc.at[pl.ds(abase, ROWS_LAST)],
                out_hbm.at[c, pl.ds(abase, ROWS_LAST)],
            )

    return seg_sum(src_flat, dst_flat, zpad, x)


def _mlp_bn_tc(x, agg2, W1, b1, W2, b2, gamma, beta, pad_count):
    def body(x_ref, agg_ref, w1_ref, b1_ref, w2_ref, b2_ref, g_ref, be_ref, out_ref):
        xv = x_ref[...]
        h = xv + agg_ref[0] + agg_ref[1]
        if pad_count:
            row = lax.broadcasted_iota(jnp.int32, (N_NODES, D), 0)
            h = h - jnp.where(row == 0, jnp.float32(pad_count), 0.0) * xv
        h = jnp.dot(h, w1_ref[...], preferred_element_type=jnp.float32) + b1_ref[...]
        h = jnp.maximum(h, 0.0)
        h = jnp.dot(h, w2_ref[...], preferred_element_type=jnp.float32) + b2_ref[...]
        h = jnp.maximum(h, 0.0)
        mean = jnp.mean(h, axis=0, keepdims=True)
        var = jnp.mean((h - mean) ** 2, axis=0, keepdims=True)
        inv = lax.rsqrt(var + BN_EPS)
        out_ref[...] = g_ref[...] * (h - mean) * inv + be_ref[...]

    return pl.pallas_call(
        body,
        out_shape=jax.ShapeDtypeStruct((N_NODES, D), jnp.float32),
    )(x, agg2, W1, b1, W2, b2, gamma, beta)


def _round6(v):
    return (v // 6) * 6


def kernel(x, edge_index, edge_attr, W1, b1, W2, b2, gamma, beta):
    del edge_attr
    src = edge_index[0].astype(jnp.int32)
    dst = edge_index[1].astype(jnp.int32)
    e = src.shape[0]
    step = NS * CHUNK * 6
    e_pad = ((e + step - 1) // step) * step
    pad = e_pad - e

    total_chunks = e_pad // CHUNK
    n_per_tile = total_chunks // NW
    n0 = _round6(int(round(n_per_tile * 2 * 0.858)))
    n1 = (total_chunks // NS) - n0
    assert n1 % 6 == 0 and n1 >= 18 and n0 >= 18

    zpad = jnp.zeros((CHUNK,), jnp.int32)
    agg2 = _segment_sum_sc(src, dst, zpad, x, n0, n1)
    return _mlp_bn_tc(
        x, agg2, W1, b1.reshape(1, D), W2, b2.reshape(1, D),
        gamma.reshape(1, D), beta.reshape(1, D), pad,
    )


# speedup vs baseline: 1.0302x; 1.0302x over previous
import functools

import jax
import jax.numpy as jnp
from jax import lax
from jax.experimental import pallas as pl
from jax.experimental.pallas import tpu as pltpu
from jax.experimental.pallas import tpu_sc as plsc

N_NODES = 10000
D = 128
BN_EPS = 1e-5

NC = 2
NS = 16
NW = NC * NS
CHUNK = 120
ROWS_MAIN = 632
ROWS_LAST = N_NODES - 15 * ROWS_MAIN


def _segment_sum_sc(src_flat, dst_flat, zpad, x, n0, n1):
    e = src_flat.shape[0]
    full = e // CHUNK
    tail = e - full * CHUNK
    mesh = plsc.VectorSubcoreMesh(core_axis_name="c", subcore_axis_name="s")

    @functools.partial(
        pl.kernel,
        mesh=mesh,
        out_type=jax.ShapeDtypeStruct((NC, N_NODES, D), jnp.float32),
        scratch_types=[
            pltpu.VMEM((6, CHUNK), jnp.int32),
            pltpu.VMEM((6, CHUNK), jnp.int32),
            pltpu.VMEM((CHUNK, D), jnp.float32),
            pltpu.VMEM((CHUNK, D), jnp.float32),
            pltpu.VMEM((CHUNK, D), jnp.float32),
            pltpu.VMEM_SHARED((N_NODES, D), jnp.float32),
            pltpu.SemaphoreType.DMA,
            pltpu.SemaphoreType.DMA,
            pltpu.SemaphoreType.DMA,
            pltpu.SemaphoreType.DMA,
            pltpu.SemaphoreType.DMA,
            pltpu.SemaphoreType.DMA,
            pltpu.SemaphoreType.DMA,
            pltpu.SemaphoreType.DMA,
        ],
    )
    def seg_sum(src_hbm, dst_hbm, zpad_hbm, x_hbm, out_hbm, sidx, didx,
                rows0, rows1, rows2, acc,
                semg0, semg1, semg2, sems0, sems1, sems2, semia, semib):
        c = lax.axis_index("c")
        s = lax.axis_index("s")
        n_chunks = jnp.where(c == 0, n0, n1)
        n_super = n_chunks // 6

        bufs = (rows0, rows1, rows2)
        semgs = (semg0, semg1, semg2)
        semss = (sems0, sems1, sems2)

        zero = jnp.zeros((16,), jnp.float32)

        def zrow(i, _):
            def zcol(j, _):
                rows0[i, pl.ds(j * 16, 16)] = zero
                return 0
            return lax.fori_loop(0, D // 16, zcol, 0)

        lax.fori_loop(0, CHUNK, zrow, 0)

        abase = s * ROWS_MAIN

        def zero_span(nrows):
            for i in range(nrows // CHUNK):
                pltpu.sync_copy(rows0, acc.at[pl.ds(abase + i * CHUNK, CHUNK)])
            rem = nrows % CHUNK
            if rem:
                pltpu.sync_copy(
                    rows0.at[pl.ds(0, rem)],
                    acc.at[pl.ds(abase + (nrows // CHUNK) * CHUNK, rem)])

        @pl.when(s < NS - 1)
        def _():
            zero_span(ROWS_MAIN)

        @pl.when(s == NS - 1)
        def _():
            zero_span(ROWS_LAST)

        cbase = jnp.where(c == 0, s * n0, NS * n0 + s * n1)

        def load_idx(arr, row_dst, ch, sem):
            @pl.when(ch < full)
            def _():
                pltpu.async_copy(arr.at[pl.ds(ch * CHUNK, CHUNK)], row_dst, sem)

            if tail:
                @pl.when(ch == full)
                def _():
                    pltpu.async_copy(
                        arr.at[pl.ds(full * CHUNK, tail)],
                        row_dst.at[pl.ds(0, tail)], sem)
                    pltpu.async_copy(
                        zpad_hbm.at[pl.ds(0, CHUNK - tail)],
                        row_dst.at[pl.ds(tail, CHUNK - tail)], sem)

            @pl.when(ch > full if tail else ch >= full)
            def _():
                pltpu.async_copy(zpad_hbm, row_dst, sem)

        def drain_idx(arr, row_dst, ch, sem):
            @pl.when(ch < full)
            def _():
                pltpu.make_async_copy(
                    arr.at[pl.ds(ch * CHUNK, CHUNK)], row_dst, sem).wait()

            if tail:
                @pl.when(ch == full)
                def _():
                    pltpu.make_async_copy(
                        arr.at[pl.ds(full * CHUNK, tail)],
                        row_dst.at[pl.ds(0, tail)], sem).wait()
                    pltpu.make_async_copy(
                        zpad_hbm.at[pl.ds(0, CHUNK - tail)],
                        row_dst.at[pl.ds(tail, CHUNK - tail)], sem).wait()

            @pl.when(ch > full if tail else ch >= full)
            def _():
                pltpu.make_async_copy(zpad_hbm, row_dst, sem).wait()

        for u in range(6):
            load_idx(src_hbm, sidx.at[u], cbase + u, semia)
            load_idx(dst_hbm, didx.at[u], cbase + u, semia)
        for u in range(6):
            drain_idx(src_hbm, sidx.at[u], cbase + u, semia)
            drain_idx(dst_hbm, didx.at[u], cbase + u, semia)
        plsc.subcore_barrier()

        def wait_g(u, b):
            pltpu.make_async_copy(x_hbm.at[sidx.at[u]], bufs[b], semgs[b]).wait()

        def issue_s(u, b):
            pltpu.async_copy(bufs[b], acc.at[didx.at[u]], semss[b], add=True)

        def wait_s(u, b):
            pltpu.make_async_copy(bufs[b], acc.at[didx.at[u]], semss[b]).wait()

        def issue_g(u, b):
            pltpu.async_copy(x_hbm.at[sidx.at[u]], bufs[b], semgs[b])

        def refill(slot0, c0, sem):
            for u in range(3):
                load_idx(src_hbm, sidx.at[slot0 + u], c0 + u, sem)
                load_idx(dst_hbm, didx.at[slot0 + u], c0 + u, sem)

        def wait_refill(slot0, c0, sem):
            for u in range(3):
                drain_idx(src_hbm, sidx.at[slot0 + u], c0 + u, sem)
                drain_idx(dst_hbm, didx.at[slot0 + u], c0 + u, sem)

        def first_half(j0, do_refill_a, wait_ib, issue_gathers):
            for u in range(3):
                wait_g(u, u)
                issue_s(u, u)
            if wait_ib:
                wait_refill(3, cbase + j0 + 3, semib)
            for u in range(3):
                wait_s(u, u)
                if issue_gathers:
                    issue_g(3 + u, u)
            if do_refill_a:
                refill(0, cbase + j0 + 6, semia)

        def second_half(j0, do_refill_b, wait_ia, issue_gathers):
            for u in range(3):
                wait_g(3 + u, u)
                issue_s(3 + u, u)
            if wait_ia:
                wait_refill(0, cbase + j0 + 6, semia)
            for u in range(3):
                wait_s(3 + u, u)
                if issue_gathers:
                    issue_g(u, u)
            if do_refill_b:
                refill(3, cbase + j0 + 9, semib)

        for u in range(3):
            issue_g(u, u)

        first_half(0, do_refill_a=True, wait_ib=False, issue_gathers=True)
        second_half(0, do_refill_b=True, wait_ia=True, issue_gathers=True)

        def body(jj, _):
            j0 = jj * 6
            first_half(j0, True, True, True)
            second_half(j0, True, True, True)
            return 0

        lax.fori_loop(1, n_super - 1, body, 0)

        j0e = (n_super - 1) * 6
        first_half(j0e, do_refill_a=False, wait_ib=True, issue_gathers=True)
        second_half(j0e, do_refill_b=False, wait_ia=False, issue_gathers=False)

        plsc.subcore_barrier()

        @pl.when(s < NS - 1)
        def _():
            pltpu.sync_copy(
                acc.at[pl.ds(abase, ROWS_MAIN)],
                out_hbm.at[c, pl.ds(abase, ROWS_MAIN)],
            )

        @pl.when(s == NS - 1)
        def _():
            pltpu.sync_copy(
                acc.at[pl.ds(abase, ROWS_LAST)],
                out_hbm.at[c, pl.ds(abase, ROWS_LAST)],
            )

    return seg_sum(src_flat, dst_flat, zpad, x)


def _mlp_bn_tc(x, agg2, W1, b1, W2, b2, gamma, beta, pad_count):
    def body(x_ref, agg_ref, w1_ref, b1_ref, w2_ref, b2_ref, g_ref, be_ref, out_ref):
        xv = x_ref[...]
        h = xv + agg_ref[0] + agg_ref[1]
        if pad_count:
            row = lax.broadcasted_iota(jnp.int32, (N_NODES, D), 0)
            h = h - jnp.where(row == 0, jnp.float32(pad_count), 0.0) * xv
        h = jnp.dot(h, w1_ref[...], preferred_element_type=jnp.float32) + b1_ref[...]
        h = jnp.maximum(h, 0.0)
        h = jnp.dot(h, w2_ref[...], preferred_element_type=jnp.float32) + b2_ref[...]
        h = jnp.maximum(h, 0.0)
        mean = jnp.mean(h, axis=0, keepdims=True)
        var = jnp.mean((h - mean) ** 2, axis=0, keepdims=True)
        inv = lax.rsqrt(var + BN_EPS)
        out_ref[...] = g_ref[...] * (h - mean) * inv + be_ref[...]

    return pl.pallas_call(
        body,
        out_shape=jax.ShapeDtypeStruct((N_NODES, D), jnp.float32),
    )(x, agg2, W1, b1, W2, b2, gamma, beta)


def _round6(v):
    return (v // 6) * 6


def kernel(x, edge_index, edge_attr, W1, b1, W2, b2, gamma, beta):
    del edge_attr
    src = edge_index[0].astype(jnp.int32)
    dst = edge_index[1].astype(jnp.int32)
    e = src.shape[0]
    step = NS * CHUNK * 6
    e_pad = ((e + step - 1) // step) * step
    pad = e_pad - e

    total_chunks = e_pad // CHUNK
    n_per_tile = total_chunks // NW
    n0 = _round6(int(round(n_per_tile * 2 * 0.822)))
    n1 = (total_chunks // NS) - n0
    assert n1 % 6 == 0 and n1 >= 18 and n0 >= 18

    zpad = jnp.zeros((CHUNK,), jnp.int32)
    agg2 = _segment_sum_sc(src, dst, zpad, x, n0, n1)
    return _mlp_bn_tc(
        x, agg2, W1, b1.reshape(1, D), W2, b2.reshape(1, D),
        gamma.reshape(1, D), beta.reshape(1, D), pad,
    )
